# Initial kernel scaffold; baseline (speedup 1.0000x reference)
import jax
import jax.numpy as jnp
from jax.experimental import pallas as pl


def kernel(inputs, emb):
    raise NotImplementedError("write your pallas kernel here")


# speedup vs baseline: 1.1410x; 1.1410x over previous
import jax
import jax.numpy as jnp
from jax.experimental import pallas as pl

_BM = 256
_BOUNDS = ((0, 4096), (4096, 8192))
_BIG = 2 ** 30


def _vq_body(c_ref, f_ref, f2_ref, c2_ref, out_ref):
    fneg = f_ref[...] * (-2.0)
    tneg = jax.lax.dot_general(
        c_ref[...], fneg, (((1,), (1,)), ((), ())),
        preferred_element_type=jnp.float32)
    d = (f2_ref[...] + tneg) + c2_ref[...]

    vq = None
    idx = None
    for lo, hi in _BOUNDS:
        seg = d[lo:hi, :]
        m = jnp.min(seg, axis=0, keepdims=True)
        io = jax.lax.broadcasted_iota(jnp.int32, seg.shape, 0)
        cand = jnp.where(seg == m, io, _BIG)
        i_ = jnp.min(cand, axis=0, keepdims=True) + lo
        if vq is None:
            vq, idx = m, i_
        else:
            take = m < vq
            idx = jnp.where(take, i_, idx)
            vq = jnp.where(take, m, vq)
        vq = vq.astype(jnp.bfloat16).astype(jnp.float32)
    out_ref[...] = idx.reshape(1, 1, _BM)


def kernel(inputs, emb):
    B, D = inputs.shape
    K, _ = emb.shape
    f2 = jnp.sum(inputs * inputs, axis=1, keepdims=True)
    c2 = jnp.sum(emb * emb, axis=1)
    min_ind = pl.pallas_call(
        _vq_body,
        grid=(B // _BM,),
        in_specs=[
            pl.BlockSpec((K, D), lambda i: (0, 0)),
            pl.BlockSpec((_BM, D), lambda i: (i, 0)),
            pl.BlockSpec((1, _BM), lambda i: (0, i)),
            pl.BlockSpec((K, 1), lambda i: (0, 0)),
        ],
        out_specs=pl.BlockSpec((1, 1, _BM), lambda i: (i, 0, 0)),
        out_shape=jax.ShapeDtypeStruct((B // _BM, 1, _BM), jnp.int32),
    )(emb, inputs, f2.reshape(1, B), c2.reshape(K, 1))
    return (inputs, emb, min_ind.reshape(B))


# speedup vs baseline: 1.6114x; 1.4123x over previous
import jax
import jax.numpy as jnp
from jax.experimental import pallas as pl

_BM = 256
_BOUNDS = ((0, 4096), (4096, 8192))
_BIG = 2 ** 30


def _vq_body(c_ref, f_ref, f2_ref, c2_ref, out_ref):
    fneg = f_ref[...] * (-2.0)
    tneg = jax.lax.dot_general(
        c_ref[...], fneg, (((1,), (1,)), ((), ())),
        preferred_element_type=jnp.float32)
    d = (f2_ref[...] + tneg) + c2_ref[...]

    vq = None
    idx = None
    for lo, hi in _BOUNDS:
        seg = d[lo:hi, :]
        m = jnp.min(seg, axis=0, keepdims=True)
        i_ = jnp.argmin(seg, axis=0).astype(jnp.int32)[None, :] + lo
        if vq is None:
            vq, idx = m, i_
        else:
            take = m < vq
            idx = jnp.where(take, i_, idx)
            vq = jnp.where(take, m, vq)
        vq = vq.astype(jnp.bfloat16).astype(jnp.float32)
    out_ref[...] = idx.reshape(1, 1, _BM)


def kernel(inputs, emb):
    B, D = inputs.shape
    K, _ = emb.shape
    f2 = jnp.sum(inputs * inputs, axis=1, keepdims=True)
    c2 = jnp.sum(emb * emb, axis=1)
    min_ind = pl.pallas_call(
        _vq_body,
        grid=(B // _BM,),
        in_specs=[
            pl.BlockSpec((K, D), lambda i: (0, 0)),
            pl.BlockSpec((_BM, D), lambda i: (i, 0)),
            pl.BlockSpec((1, _BM), lambda i: (0, i)),
            pl.BlockSpec((K, 1), lambda i: (0, 0)),
        ],
        out_specs=pl.BlockSpec((1, 1, _BM), lambda i: (i, 0, 0)),
        out_shape=jax.ShapeDtypeStruct((B // _BM, 1, _BM), jnp.int32),
    )(emb, inputs, f2.reshape(1, B), c2.reshape(K, 1))
    return (inputs, emb, min_ind.reshape(B))
